# baseline (speedup 1.0000x reference)
import functools

import jax
import jax.numpy as jnp
from jax.experimental import pallas as pl
from jax.experimental.pallas import tpu as pltpu

H = 12
M_LANDMARKS = 256
TOPK = 4
NEG = -1e30
BIG = 1024.0
BF = jnp.bfloat16


def _mm(a, b, dims):
    return jax.lax.dot_general(a.astype(BF), b.astype(BF), (dims, ((), ())),
                               preferred_element_type=jnp.float32)


def _qkv_kernel(x_ref, w_ref, o_ref):
    o_ref[...] = _mm(x_ref[...], w_ref[...], ((1,), (1,)))


def _attn_kernel(q_ref, k_ref, v_ref, r_ref, wp_ref, b_ref, o_ref, cent_ref,
                 *, seg, scale, hd):
    i = pl.program_id(0)
    hp = pl.program_id(1)
    bq = q_ref.shape[0]
    n = k_ref.shape[0]
    m = n // seg
    lane_m = jax.lax.broadcasted_iota(jnp.int32, (bq, m), 1)
    rbf = r_ref[...]

    @pl.when(i == 0)
    def _():
        w = k_ref.shape[1]
        cent_ref[hp] = jnp.mean(k_ref[...].reshape(m, seg, w), axis=1)

    @pl.when(hp == 0)
    def _():
        o_ref[...] = jnp.broadcast_to(b_ref[...], o_ref.shape)

    cent2 = cent_ref[hp]
    o_halves = []
    for half in range(2):
        sl = slice(half * hd, (half + 1) * hd)
        q = q_ref[:, sl]
        k = k_ref[:, sl]
        v = v_ref[:, sl]

        rs = _mm(q, cent2[:, sl], ((1,), (1,)))

        masked = rs
        selbig = jnp.zeros((bq, m), dtype=jnp.float32)
        for _ in range(TOPK):
            mx = jnp.max(masked, axis=1, keepdims=True)
            eq = masked == mx
            idx = jnp.min(jnp.where(eq, lane_m, m), axis=1, keepdims=True)
            hit = lane_m == idx
            masked = jnp.where(hit, NEG, masked)
            selbig = jnp.where(hit, BIG, selbig)

        s = _mm(q * scale, k, ((1,), (1,)))
        s = s + _mm(selbig, rbf, ((1,), (1,)))
        mxs = jnp.max(s, axis=1, keepdims=True)
        e = jnp.exp(s - mxs)
        p = e * (1.0 / jnp.sum(e, axis=1, keepdims=True))
        o_halves.append(_mm(p, v, ((1,), (0,))))

    o_pair = jnp.concatenate(o_halves, axis=1)
    o_ref[...] += _mm(o_pair, wp_ref[...], ((1,), (1,)))


@functools.partial(jax.jit, static_argnames=("interpret",))
def kernel(x, W_qkv, W_proj, b_proj, interpret=False):
    Bb, Nn, Cc = x.shape
    hd = Cc // H
    scale = hd ** (-0.5)
    m = min(M_LANDMARKS, Nn)
    seg = (Nn + m - 1) // m

    xf = x.reshape(Bb * Nn, Cc)
    bn = Bb * Nn
    blk = 256
    grid_a = (bn // blk,)

    qkv = pl.pallas_call(
        _qkv_kernel,
        grid=grid_a,
        in_specs=[
            pl.BlockSpec((blk, Cc), lambda i: (i, 0)),
            pl.BlockSpec((3 * Cc, Cc), lambda i: (0, 0)),
        ],
        out_specs=pl.BlockSpec((blk, 3 * Cc), lambda i: (i, 0)),
        out_shape=jax.ShapeDtypeStruct((bn, 3 * Cc), jnp.float32),
        interpret=interpret,
    )(xf, W_qkv)

    rbf = (jnp.arange(Nn, dtype=jnp.int32)[:, None] // seg
           == jnp.arange(m, dtype=jnp.int32)[None, :]).astype(BF)

    bq = 256
    hp = H // 2
    grid_b = (Nn // bq, hp)
    out = pl.pallas_call(
        functools.partial(_attn_kernel, seg=seg, scale=scale, hd=hd),
        grid=grid_b,
        in_specs=[
            pl.BlockSpec((bq, 2 * hd), lambda i, h: (i, h)),
            pl.BlockSpec((Nn, 2 * hd), lambda i, h: (0, hp + h)),
            pl.BlockSpec((Nn, 2 * hd), lambda i, h: (0, 2 * hp + h)),
            pl.BlockSpec((Nn, m), lambda i, h: (0, 0)),
            pl.BlockSpec((Cc, 2 * hd), lambda i, h: (0, h)),
            pl.BlockSpec((1, Cc), lambda i, h: (0, 0)),
        ],
        out_specs=pl.BlockSpec((bq, Cc), lambda i, h: (i, 0)),
        out_shape=jax.ShapeDtypeStruct((bn, Cc), jnp.float32),
        scratch_shapes=[pltpu.VMEM((hp, m, 2 * hd), jnp.float32)],
        interpret=interpret,
    )(qkv, qkv, qkv, rbf, W_proj, b_proj.reshape(1, Cc))

    return out.reshape(Bb, Nn, Cc)


# speedup vs baseline: 1.3053x; 1.3053x over previous
import functools

import jax
import jax.numpy as jnp
from jax.experimental import pallas as pl
from jax.experimental.pallas import tpu as pltpu

H = 12
M_LANDMARKS = 256
TOPK = 4
NEG = -1e30
BIG = 1024.0
BF = jnp.bfloat16


def _mm(a, b, dims):
    return jax.lax.dot_general(a.astype(BF), b.astype(BF), (dims, ((), ())),
                               preferred_element_type=jnp.float32)


def _qkv_kernel(x_ref, w_ref, o_ref):
    o_ref[...] = _mm(x_ref[...], w_ref[...], ((1,), (1,)))


def _attn_kernel(q_ref, k_ref, v_ref, r_ref, wp_ref, b_ref, o_ref, cent_ref,
                 *, seg, scale, hd):
    i = pl.program_id(0)
    hp = pl.program_id(1)
    bq = q_ref.shape[0]
    n = k_ref.shape[0]
    m = n // seg
    lane_m = jax.lax.broadcasted_iota(jnp.int32, (bq, m), 1)
    rbf = r_ref[...]

    @pl.when(i == 0)
    def _():
        w = k_ref.shape[1]
        cent_ref[hp] = jnp.mean(k_ref[...].reshape(m, seg, w), axis=1)

    @pl.when(hp == 0)
    def _():
        o_ref[...] = jnp.broadcast_to(b_ref[...], o_ref.shape)

    cent2 = cent_ref[hp]
    o_halves = []
    for half in range(2):
        sl = slice(half * hd, (half + 1) * hd)
        q = q_ref[:, sl]
        k = k_ref[:, sl]
        v = v_ref[:, sl]

        rs = _mm(q, cent2[:, sl], ((1,), (1,)))

        masked = rs
        selbig = jnp.zeros((bq, m), dtype=jnp.float32)
        for _ in range(TOPK):
            mx = jnp.max(masked, axis=1, keepdims=True)
            eq = masked == mx
            idx = jnp.min(jnp.where(eq, lane_m, m), axis=1, keepdims=True)
            hit = lane_m == idx
            masked = jnp.where(hit, NEG, masked)
            selbig = jnp.where(hit, BIG, selbig)

        s = _mm(q * scale, k, ((1,), (1,)))
        s = s + _mm(selbig, rbf, ((1,), (1,)))
        mxs = jnp.max(s, axis=1, keepdims=True)
        e = jnp.exp(s - mxs)
        p = e * (1.0 / jnp.sum(e, axis=1, keepdims=True))
        o_halves.append(_mm(p, v, ((1,), (0,))))

    o_pair = jnp.concatenate(o_halves, axis=1)
    o_ref[...] += _mm(o_pair, wp_ref[...], ((1,), (1,)))


@functools.partial(jax.jit, static_argnames=("interpret",))
def kernel(x, W_qkv, W_proj, b_proj, interpret=False):
    Bb, Nn, Cc = x.shape
    hd = Cc // H
    scale = hd ** (-0.5)
    m = min(M_LANDMARKS, Nn)
    seg = (Nn + m - 1) // m

    xf = x.reshape(Bb * Nn, Cc)
    bn = Bb * Nn
    blk = 256
    grid_a = (bn // blk,)

    qkv = pl.pallas_call(
        _qkv_kernel,
        grid=grid_a,
        in_specs=[
            pl.BlockSpec((blk, Cc), lambda i: (i, 0)),
            pl.BlockSpec((3 * Cc, Cc), lambda i: (0, 0)),
        ],
        out_specs=pl.BlockSpec((blk, 3 * Cc), lambda i: (i, 0)),
        out_shape=jax.ShapeDtypeStruct((bn, 3 * Cc), jnp.float32),
        interpret=interpret,
    )(xf, W_qkv)

    rbf = (jnp.arange(Nn, dtype=jnp.int32)[:, None] // seg
           == jnp.arange(m, dtype=jnp.int32)[None, :]).astype(BF)

    bq = 512
    hp = H // 2
    grid_b = (Nn // bq, hp)
    out = pl.pallas_call(
        functools.partial(_attn_kernel, seg=seg, scale=scale, hd=hd),
        grid=grid_b,
        in_specs=[
            pl.BlockSpec((bq, 2 * hd), lambda i, h: (i, h)),
            pl.BlockSpec((Nn, 2 * hd), lambda i, h: (0, hp + h)),
            pl.BlockSpec((Nn, 2 * hd), lambda i, h: (0, 2 * hp + h)),
            pl.BlockSpec((Nn, m), lambda i, h: (0, 0)),
            pl.BlockSpec((Cc, 2 * hd), lambda i, h: (0, h)),
            pl.BlockSpec((1, Cc), lambda i, h: (0, 0)),
        ],
        out_specs=pl.BlockSpec((bq, Cc), lambda i, h: (i, 0)),
        out_shape=jax.ShapeDtypeStruct((bn, Cc), jnp.float32),
        scratch_shapes=[pltpu.VMEM((hp, m, 2 * hd), jnp.float32)],
        interpret=interpret,
    )(qkv, qkv, qkv, rbf, W_proj, b_proj.reshape(1, Cc))

    return out.reshape(Bb, Nn, Cc)


# speedup vs baseline: 1.4419x; 1.1046x over previous
import functools

import jax
import jax.numpy as jnp
from jax.experimental import pallas as pl
from jax.experimental.pallas import tpu as pltpu

H = 12
M_LANDMARKS = 256
TOPK = 4
NEG = -1e30
BIG = 1024.0
BF = jnp.bfloat16


def _mm(a, b, dims):
    return jax.lax.dot_general(a.astype(BF), b.astype(BF), (dims, ((), ())),
                               preferred_element_type=jnp.float32)


def _qkv_kernel(x_ref, w_ref, o_ref):
    o_ref[...] = _mm(x_ref[...], w_ref[...], ((1,), (1,)))


def _attn_kernel(q_ref, k_ref, v_ref, r_ref, wp_ref, b_ref, o_ref, cent_ref,
                 *, seg, scale, hd):
    i = pl.program_id(0)
    hp = pl.program_id(1)
    bq = q_ref.shape[0]
    n = k_ref.shape[0]
    m = n // seg
    lane_m = jax.lax.broadcasted_iota(jnp.int32, (bq, m), 1)
    rbf = r_ref[...]

    @pl.when(i == 0)
    def _():
        w = k_ref.shape[1]
        cent_ref[hp] = jnp.mean(k_ref[...].reshape(m, seg, w), axis=1)

    @pl.when(hp == 0)
    def _():
        o_ref[...] = jnp.broadcast_to(b_ref[...], o_ref.shape)

    cent2 = cent_ref[hp]
    o_halves = []
    for half in range(2):
        sl = slice(half * hd, (half + 1) * hd)
        q = q_ref[:, sl]
        k = k_ref[:, sl]
        v = v_ref[:, sl]

        rs = _mm(q, cent2[:, sl], ((1,), (1,)))

        masked = rs
        selbig = jnp.zeros((bq, m), dtype=jnp.float32)
        for _ in range(TOPK):
            mx = jnp.max(masked, axis=1, keepdims=True)
            eq = masked == mx
            idx = jnp.min(jnp.where(eq, lane_m, m), axis=1, keepdims=True)
            hit = lane_m == idx
            masked = jnp.where(hit, NEG, masked)
            selbig = jnp.where(hit, BIG, selbig)

        s = _mm(q * scale, k, ((1,), (1,)))
        s = s + _mm(selbig, rbf, ((1,), (1,)))
        mxs = jnp.max(s, axis=1, keepdims=True)
        e = jnp.exp(s - mxs)
        p = e * (1.0 / jnp.sum(e, axis=1, keepdims=True))
        o_halves.append(_mm(p, v, ((1,), (0,))))

    o_pair = jnp.concatenate(o_halves, axis=1)
    o_ref[...] += _mm(o_pair, wp_ref[...], ((1,), (1,)))


@functools.partial(jax.jit, static_argnames=("interpret",))
def kernel(x, W_qkv, W_proj, b_proj, interpret=False):
    Bb, Nn, Cc = x.shape
    hd = Cc // H
    scale = hd ** (-0.5)
    m = min(M_LANDMARKS, Nn)
    seg = (Nn + m - 1) // m

    xf = x.reshape(Bb * Nn, Cc)
    bn = Bb * Nn
    blk = 256
    grid_a = (bn // blk,)

    qkv = pl.pallas_call(
        _qkv_kernel,
        grid=grid_a,
        in_specs=[
            pl.BlockSpec((blk, Cc), lambda i: (i, 0)),
            pl.BlockSpec((3 * Cc, Cc), lambda i: (0, 0)),
        ],
        out_specs=pl.BlockSpec((blk, 3 * Cc), lambda i: (i, 0)),
        out_shape=jax.ShapeDtypeStruct((bn, 3 * Cc), jnp.float32),
        interpret=interpret,
    )(xf, W_qkv)

    rbf = (jnp.arange(Nn, dtype=jnp.int32)[:, None] // seg
           == jnp.arange(m, dtype=jnp.int32)[None, :]).astype(BF)

    bq = 1024
    hp = H // 2
    grid_b = (Nn // bq, hp)
    out = pl.pallas_call(
        functools.partial(_attn_kernel, seg=seg, scale=scale, hd=hd),
        grid=grid_b,
        in_specs=[
            pl.BlockSpec((bq, 2 * hd), lambda i, h: (i, h)),
            pl.BlockSpec((Nn, 2 * hd), lambda i, h: (0, hp + h)),
            pl.BlockSpec((Nn, 2 * hd), lambda i, h: (0, 2 * hp + h)),
            pl.BlockSpec((Nn, m), lambda i, h: (0, 0)),
            pl.BlockSpec((Cc, 2 * hd), lambda i, h: (0, h)),
            pl.BlockSpec((1, Cc), lambda i, h: (0, 0)),
        ],
        out_specs=pl.BlockSpec((bq, Cc), lambda i, h: (i, 0)),
        out_shape=jax.ShapeDtypeStruct((bn, Cc), jnp.float32),
        scratch_shapes=[pltpu.VMEM((hp, m, 2 * hd), jnp.float32)],
        interpret=interpret,
    )(qkv, qkv, qkv, rbf, W_proj, b_proj.reshape(1, Cc))

    return out.reshape(Bb, Nn, Cc)


# speedup vs baseline: 1.4733x; 1.0218x over previous
import functools

import jax
import jax.numpy as jnp
from jax.experimental import pallas as pl
from jax.experimental.pallas import tpu as pltpu

H = 12
M_LANDMARKS = 256
TOPK = 4
NEG = -1e30
BIG = 1024.0
BF = jnp.bfloat16


def _mm(a, b, dims):
    return jax.lax.dot_general(a.astype(BF), b.astype(BF), (dims, ((), ())),
                               preferred_element_type=jnp.float32)


def _qkv_kernel(x_ref, w_ref, o_ref):
    o_ref[...] = _mm(x_ref[...], w_ref[...], ((1,), (1,)))


def _attn_kernel(q_ref, k_ref, v_ref, r_ref, wp_ref, b_ref, o_ref, cent_ref,
                 *, seg, scale, hd):
    i = pl.program_id(0)
    hp = pl.program_id(1)
    bq = q_ref.shape[0]
    n = k_ref.shape[0]
    m = n // seg
    lane_m = jax.lax.broadcasted_iota(jnp.int32, (bq, m), 1)
    rbf = r_ref[...]

    @pl.when(i == 0)
    def _():
        w = k_ref.shape[1]
        cent_ref[hp] = jnp.mean(k_ref[...].reshape(m, seg, w), axis=1)

    @pl.when(hp == 0)
    def _():
        o_ref[...] = jnp.broadcast_to(b_ref[...], o_ref.shape)

    cent2 = cent_ref[hp]
    o_halves = []
    for half in range(2):
        sl = slice(half * hd, (half + 1) * hd)
        q = q_ref[:, sl]
        k = k_ref[:, sl]
        v = v_ref[:, sl]

        rs = _mm(q, cent2[:, sl], ((1,), (1,)))

        masked = rs
        selbig = jnp.zeros((bq, m), dtype=jnp.float32)
        for _ in range(TOPK):
            mx = jnp.max(masked, axis=1, keepdims=True)
            eq = masked == mx
            idx = jnp.min(jnp.where(eq, lane_m, m), axis=1, keepdims=True)
            hit = lane_m == idx
            masked = jnp.where(hit, NEG, masked)
            selbig = jnp.where(hit, BIG, selbig)

        s = _mm(q * scale, k, ((1,), (1,)))
        s = s + _mm(selbig, rbf, ((1,), (1,)))
        mxs = jnp.max(s, axis=1, keepdims=True)
        e = jnp.exp(s - mxs)
        p = e * (1.0 / jnp.sum(e, axis=1, keepdims=True))
        o_halves.append(_mm(p, v, ((1,), (0,))))

    o_pair = jnp.concatenate(o_halves, axis=1)
    o_ref[...] += _mm(o_pair, wp_ref[...], ((1,), (1,)))


@functools.partial(jax.jit, static_argnames=("interpret",))
def kernel(x, W_qkv, W_proj, b_proj, interpret=False):
    Bb, Nn, Cc = x.shape
    hd = Cc // H
    scale = hd ** (-0.5)
    m = min(M_LANDMARKS, Nn)
    seg = (Nn + m - 1) // m

    xf = x.reshape(Bb * Nn, Cc)
    bn = Bb * Nn
    blk = 256
    grid_a = (bn // blk,)

    qkv = pl.pallas_call(
        _qkv_kernel,
        grid=grid_a,
        in_specs=[
            pl.BlockSpec((blk, Cc), lambda i: (i, 0)),
            pl.BlockSpec((3 * Cc, Cc), lambda i: (0, 0)),
        ],
        out_specs=pl.BlockSpec((blk, 3 * Cc), lambda i: (i, 0)),
        out_shape=jax.ShapeDtypeStruct((bn, 3 * Cc), jnp.float32),
        interpret=interpret,
    )(xf, W_qkv)

    rbf = (jnp.arange(Nn, dtype=jnp.int32)[:, None] // seg
           == jnp.arange(m, dtype=jnp.int32)[None, :]).astype(BF)

    bq = 2048
    hp = H // 2
    grid_b = (Nn // bq, hp)
    out = pl.pallas_call(
        functools.partial(_attn_kernel, seg=seg, scale=scale, hd=hd),
        grid=grid_b,
        in_specs=[
            pl.BlockSpec((bq, 2 * hd), lambda i, h: (i, h)),
            pl.BlockSpec((Nn, 2 * hd), lambda i, h: (0, hp + h)),
            pl.BlockSpec((Nn, 2 * hd), lambda i, h: (0, 2 * hp + h)),
            pl.BlockSpec((Nn, m), lambda i, h: (0, 0)),
            pl.BlockSpec((Cc, 2 * hd), lambda i, h: (0, h)),
            pl.BlockSpec((1, Cc), lambda i, h: (0, 0)),
        ],
        out_specs=pl.BlockSpec((bq, Cc), lambda i, h: (i, 0)),
        out_shape=jax.ShapeDtypeStruct((bn, Cc), jnp.float32),
        scratch_shapes=[pltpu.VMEM((hp, m, 2 * hd), jnp.float32)],
        interpret=interpret,
    )(qkv, qkv, qkv, rbf, W_proj, b_proj.reshape(1, Cc))

    return out.reshape(Bb, Nn, Cc)
